# baseline (speedup 1.0000x reference)
import functools

import jax
import jax.numpy as jnp
from jax import lax
from jax.experimental import pallas as pl
from jax.experimental.pallas import tpu as pltpu
from jax.experimental.pallas import tpu_sc as plsc

F32 = jnp.float32
NGRAPH = 256
H = 128
HH = 64
ROWBLK = 2000


def _proj_body(x_ref, w_ref, b_ref, g_ref, bt_ref, x0_ref, h_ref):
    x0 = jnp.dot(x_ref[...], w_ref[...], preferred_element_type=F32) + b_ref[...]
    x0_ref[...] = x0
    mu = jnp.mean(x0, axis=1, keepdims=True)
    var = jnp.mean((x0 - mu) ** 2, axis=1, keepdims=True)
    h = (x0 - mu) * lax.rsqrt(var + 1e-5) * g_ref[...] + bt_ref[...]
    h = jnp.maximum(h, 0.0)
    h_ref[...] = jnp.stack([h[:, :HH], h[:, HH:]], axis=0)


def _proj(x, node_W, node_b, g0, b0):
    n = x.shape[0]
    grid = n // ROWBLK
    return pl.pallas_call(
        _proj_body,
        grid=(grid,),
        in_specs=[
            pl.BlockSpec((ROWBLK, H), lambda i: (i, 0)),
            pl.BlockSpec((H, H), lambda i: (0, 0)),
            pl.BlockSpec((1, H), lambda i: (0, 0)),
            pl.BlockSpec((1, H), lambda i: (0, 0)),
            pl.BlockSpec((1, H), lambda i: (0, 0)),
        ],
        out_specs=[
            pl.BlockSpec((ROWBLK, H), lambda i: (i, 0)),
            pl.BlockSpec((2, ROWBLK, HH), lambda i: (0, i, 0)),
        ],
        out_shape=[
            jax.ShapeDtypeStruct((n, H), F32),
            jax.ShapeDtypeStruct((2, n, HH), F32),
        ],
    )(x, node_W, node_b.reshape(1, H), g0.reshape(1, H), b0.reshape(1, H))


def _edge_proj_body(a_ref, w_ref, b_ref, e_ref):
    e = jnp.dot(a_ref[...], w_ref[...], preferred_element_type=F32) + b_ref[...]
    e_ref[...] = jnp.stack([e[:, :HH], e[:, HH:]], axis=0)


def _edge_proj(edge_attr, edge_W, edge_b):
    e_num, de = edge_attr.shape
    grid = e_num // ROWBLK
    return pl.pallas_call(
        _edge_proj_body,
        grid=(grid,),
        in_specs=[
            pl.BlockSpec((ROWBLK, de), lambda i: (i, 0)),
            pl.BlockSpec((de, H), lambda i: (0, 0)),
            pl.BlockSpec((1, H), lambda i: (0, 0)),
        ],
        out_specs=pl.BlockSpec((2, ROWBLK, HH), lambda i: (0, i, 0)),
        out_shape=jax.ShapeDtypeStruct((2, e_num, HH), F32),
    )(edge_attr, edge_W, edge_b.reshape(1, H))


def _dense_body(x_ref, o0_ref, o1_ref, w_ref, b_ref, g_ref, bt_ref,
                xn_ref, h_ref):
    o0 = o0_ref[0]
    o1 = o1_ref[0]
    a0 = o0[:, :HH] / (o0[:, HH:] + 1e-16)
    a1 = o1[:, :HH] / (o1[:, HH:] + 1e-16)
    xn = (x_ref[...]
          + jnp.dot(a0, w_ref[0], preferred_element_type=F32)
          + jnp.dot(a1, w_ref[1], preferred_element_type=F32)
          + b_ref[...])
    xn_ref[...] = xn
    mu = jnp.mean(xn, axis=1, keepdims=True)
    var = jnp.mean((xn - mu) ** 2, axis=1, keepdims=True)
    h = (xn - mu) * lax.rsqrt(var + 1e-5) * g_ref[...] + bt_ref[...]
    h = jnp.maximum(h, 0.0)
    h_ref[...] = jnp.stack([h[:, :HH], h[:, HH:]], axis=0)


def _dense_step(x, accum, w2, b, g_next, b_next):
    n = x.shape[0]
    grid = n // ROWBLK
    return pl.pallas_call(
        _dense_body,
        grid=(grid,),
        in_specs=[
            pl.BlockSpec((ROWBLK, H), lambda i: (i, 0)),
            pl.BlockSpec((1, ROWBLK, H), lambda i: (0, i, 0)),
            pl.BlockSpec((1, ROWBLK, H), lambda i: (1, i, 0)),
            pl.BlockSpec((2, HH, H), lambda i: (0, 0, 0)),
            pl.BlockSpec((1, H), lambda i: (0, 0)),
            pl.BlockSpec((1, H), lambda i: (0, 0)),
            pl.BlockSpec((1, H), lambda i: (0, 0)),
        ],
        out_specs=[
            pl.BlockSpec((ROWBLK, H), lambda i: (i, 0)),
            pl.BlockSpec((2, ROWBLK, HH), lambda i: (0, i, 0)),
        ],
        out_shape=[
            jax.ShapeDtypeStruct((n, H), F32),
            jax.ShapeDtypeStruct((2, n, HH), F32),
        ],
    )(x, accum, accum, w2, b.reshape(1, H), g_next.reshape(1, H),
      b_next.reshape(1, H))


def _head_body(p1_ref, p2_ref, w1_ref, b1_ref, g1_ref, t1_ref,
               w2_ref, b2_ref, g2_ref, t2_ref, w3_ref, b3_ref, o_ref):
    def mean_pool(p):
        acc = p[0] + p[1]
        s = acc[:, :H]
        cnt = acc[:, H:]
        return s / jnp.maximum(cnt, 1.0)

    m1 = mean_pool(p1_ref[...])
    m2 = mean_pool(p2_ref[...])
    hh = (jnp.dot(m1, w1_ref[0], preferred_element_type=F32)
          + jnp.dot(m2, w1_ref[1], preferred_element_type=F32)
          + b1_ref[...])
    hh = jnp.maximum(hh * g1_ref[...] + t1_ref[...], 0.0)
    h2 = jnp.dot(hh, w2_ref[...], preferred_element_type=F32) + b2_ref[...]
    h2 = jnp.maximum(h2 * g2_ref[...] + t2_ref[...], 0.0)
    o = jnp.dot(h2, w3_ref[...], preferred_element_type=F32) + b3_ref[...]
    o_ref[...] = jax.nn.sigmoid(o)


def _head(p1, p2, fc1_W, fc1_b, bn1_g, bn1_b, fc2_W, fc2_b, bn2_g, bn2_b,
          out_W, out_b):
    full = lambda *s: pl.BlockSpec(s, lambda: tuple(0 for _ in s))
    return pl.pallas_call(
        _head_body,
        in_specs=[
            full(2, NGRAPH, 2 * H),
            full(2, NGRAPH, 2 * H),
            full(2, H, H),
            full(1, H), full(1, H), full(1, H),
            full(H, 32), full(1, 32), full(1, 32), full(1, 32),
            full(32, 1), full(1, 1),
        ],
        out_specs=full(NGRAPH, 1),
        out_shape=jax.ShapeDtypeStruct((NGRAPH, 1), F32),
    )(p1, p2, fc1_W.reshape(2, H, H), fc1_b.reshape(1, H),
      bn1_g.reshape(1, H), bn1_b.reshape(1, H), fc2_W, fc2_b.reshape(1, 32),
      bn2_g.reshape(1, 32), bn2_b.reshape(1, 32), out_W, out_b.reshape(1, 1))


EDGE_BLK = 160
NTILE = 16
ZROWS = 8
NPAD = 10240


def _edge_sc_body(h_ref, e_ref, src_ref, dst_ref, out_ref,
                  accum, srcv, dstv, hrows, erows, vals, zbuf, sem):
    c = lax.axis_index("c")
    s = lax.axis_index("s")
    n = h_ref.shape[0] // 2
    e_total = src_ref.shape[0]
    per_tile = e_total // NTILE
    nblk = per_tile // EDGE_BLK
    rows_per_tile = NPAD // NTILE

    def _zb(r, _):
        for k in range(HH // 16):
            zbuf[r, pl.ds(k * 16, 16)] = jnp.zeros((16,), F32)
            zbuf[r, pl.ds(HH + k * 16, 16)] = jnp.zeros((16,), F32)
        return 0
    lax.fori_loop(0, ZROWS, _zb, 0)
    def _zc(j, _):
        pltpu.sync_copy(zbuf, accum.at[pl.ds(s * rows_per_tile + j * ZROWS,
                                             ZROWS)])
        return 0
    lax.fori_loop(0, rows_per_tile // ZROWS, _zc, 0)
    plsc.subcore_barrier()

    def _block(k, _):
        off = s * per_tile + k * EDGE_BLK
        pltpu.sync_copy(src_ref.at[pl.ds(off, EDGE_BLK)], srcv)
        pltpu.sync_copy(dst_ref.at[pl.ds(off, EDGE_BLK)], dstv)

        def _adj(i, _):
            srcv[pl.ds(i * 16, 16)] = srcv[pl.ds(i * 16, 16)] + c * n
            return 0
        lax.fori_loop(0, EDGE_BLK // 16, _adj, 0)

        pltpu.async_copy(h_ref.at[srcv], hrows, sem).wait()
        pltpu.sync_copy(e_ref.at[pl.ds(c * e_total + off, EDGE_BLK)], erows)

        def _row(b, _):
            for k2 in range(HH // 16):
                hv = hrows[b, pl.ds(k2 * 16, 16)]
                ev = erows[b, pl.ds(k2 * 16, 16)]
                m = jnp.maximum(hv + ev, 0.0) + 1e-7
                x = jnp.exp(m)
                vals[b, pl.ds(k2 * 16, 16)] = x * m
                vals[b, pl.ds(HH + k2 * 16, 16)] = x
            return 0
        lax.fori_loop(0, EDGE_BLK, _row, 0)

        pltpu.sync_copy(vals, accum.at[dstv], add=True)
        return 0

    lax.fori_loop(0, nblk, _block, 0)
    plsc.subcore_barrier()
    pltpu.sync_copy(accum.at[pl.ds(s * rows_per_tile, rows_per_tile)],
                    out_ref.at[pl.ds(c * NPAD + s * rows_per_tile,
                                     rows_per_tile)])


def _edge_pass(h_split, e2, src, dst, n):
    h2 = h_split.reshape(2 * n, HH)
    mesh = plsc.VectorSubcoreMesh(core_axis_name="c", subcore_axis_name="s")
    out = pl.kernel(
        _edge_sc_body,
        mesh=mesh,
        compiler_params=pltpu.CompilerParams(use_tc_tiling_on_sc=False),
        out_type=jax.ShapeDtypeStruct((2 * NPAD, H), F32),
        scratch_types=[
            pltpu.VMEM_SHARED((NPAD, H), F32),
            pltpu.VMEM((EDGE_BLK,), jnp.int32),
            pltpu.VMEM((EDGE_BLK,), jnp.int32),
            pltpu.VMEM((EDGE_BLK, HH), F32),
            pltpu.VMEM((EDGE_BLK, HH), F32),
            pltpu.VMEM((EDGE_BLK, H), F32),
            pltpu.VMEM((ZROWS, H), F32),
            pltpu.SemaphoreType.DMA,
        ],
    )(h2, e2, src, dst)
    return out.reshape(2, NPAD, H)


def _pool(x, batch):
    s = jax.ops.segment_sum(x, batch, num_segments=NGRAPH)
    cnt = jax.ops.segment_sum(jnp.ones((x.shape[0],), F32), batch,
                              num_segments=NGRAPH)
    cntb = jnp.broadcast_to(cnt[:, None], (NGRAPH, H))
    part = jnp.concatenate([s, cntb], axis=1)
    return jnp.stack([part, jnp.zeros_like(part)])


def kernel(entry1_x, entry1_edge_index, entry1_edge_attr, entry1_batch,
           entry2_x, entry2_edge_index, entry2_edge_attr, entry2_batch,
           node_W, node_b, edge_W, edge_b, ln_g, ln_b, mlp_W, mlp_b,
           fc1_W, fc1_b, bn1_g, bn1_b, fc2_W, fc2_b, bn2_g, bn2_b,
           out_W, out_b):
    num_layers = mlp_W.shape[0]
    n = entry1_x.shape[0]
    e_num = entry1_edge_index.shape[1]
    src1, dst1 = entry1_edge_index[0], entry1_edge_index[1]
    src2, dst2 = entry2_edge_index[0], entry2_edge_index[1]
    e1 = _edge_proj(entry1_edge_attr, edge_W, edge_b).reshape(2 * e_num, HH)
    e2 = _edge_proj(entry2_edge_attr, edge_W, edge_b).reshape(2 * e_num, HH)
    x1, h1 = _proj(entry1_x, node_W, node_b, ln_g[0], ln_b[0])
    x2, h2 = _proj(entry2_x, node_W, node_b, ln_g[0], ln_b[0])
    for l in range(num_layers):
        a1 = _edge_pass(h1, e1, src1, dst1, n)
        a2 = _edge_pass(h2, e2, src2, dst2, n)
        nl = min(l + 1, num_layers - 1)
        w2 = mlp_W[l].reshape(2, HH, H)
        x1, h1 = _dense_step(x1, a1, w2, mlp_b[l], ln_g[nl], ln_b[nl])
        x2, h2 = _dense_step(x2, a2, w2, mlp_b[l], ln_g[nl], ln_b[nl])
    p1 = _pool(x1, entry1_batch)
    p2 = _pool(x2, entry2_batch)
    return _head(p1, p2, fc1_W, fc1_b, bn1_g, bn1_b, fc2_W, fc2_b,
                 bn2_g, bn2_b, out_W, out_b)


# speedup vs baseline: 1.2497x; 1.2497x over previous
import functools

import jax
import jax.numpy as jnp
from jax import lax
from jax.experimental import pallas as pl
from jax.experimental.pallas import tpu as pltpu
from jax.experimental.pallas import tpu_sc as plsc

F32 = jnp.float32
NGRAPH = 256
H = 128
HH = 64
ROWBLK = 2000


def _proj_body(x_ref, w_ref, b_ref, g_ref, bt_ref, x0_ref, h_ref):
    x0 = jnp.dot(x_ref[...], w_ref[...], preferred_element_type=F32) + b_ref[...]
    x0_ref[...] = x0
    mu = jnp.mean(x0, axis=1, keepdims=True)
    var = jnp.mean((x0 - mu) ** 2, axis=1, keepdims=True)
    h = (x0 - mu) * lax.rsqrt(var + 1e-5) * g_ref[...] + bt_ref[...]
    h = jnp.maximum(h, 0.0)
    h_ref[...] = jnp.stack([h[:, :HH], h[:, HH:]], axis=0)


def _proj(x, node_W, node_b, g0, b0):
    n = x.shape[0]
    grid = n // ROWBLK
    return pl.pallas_call(
        _proj_body,
        grid=(grid,),
        in_specs=[
            pl.BlockSpec((ROWBLK, H), lambda i: (i, 0)),
            pl.BlockSpec((H, H), lambda i: (0, 0)),
            pl.BlockSpec((1, H), lambda i: (0, 0)),
            pl.BlockSpec((1, H), lambda i: (0, 0)),
            pl.BlockSpec((1, H), lambda i: (0, 0)),
        ],
        out_specs=[
            pl.BlockSpec((ROWBLK, H), lambda i: (i, 0)),
            pl.BlockSpec((2, ROWBLK, HH), lambda i: (0, i, 0)),
        ],
        out_shape=[
            jax.ShapeDtypeStruct((n, H), F32),
            jax.ShapeDtypeStruct((2, n, HH), F32),
        ],
    )(x, node_W, node_b.reshape(1, H), g0.reshape(1, H), b0.reshape(1, H))


def _edge_proj_body(a_ref, w_ref, b_ref, e_ref):
    e = jnp.dot(a_ref[...], w_ref[...], preferred_element_type=F32) + b_ref[...]
    e_ref[...] = jnp.stack([e[:, :HH], e[:, HH:]], axis=0)


def _edge_proj(edge_attr, edge_W, edge_b):
    e_num, de = edge_attr.shape
    grid = e_num // ROWBLK
    return pl.pallas_call(
        _edge_proj_body,
        grid=(grid,),
        in_specs=[
            pl.BlockSpec((ROWBLK, de), lambda i: (i, 0)),
            pl.BlockSpec((de, H), lambda i: (0, 0)),
            pl.BlockSpec((1, H), lambda i: (0, 0)),
        ],
        out_specs=pl.BlockSpec((2, ROWBLK, HH), lambda i: (0, i, 0)),
        out_shape=jax.ShapeDtypeStruct((2, e_num, HH), F32),
    )(edge_attr, edge_W, edge_b.reshape(1, H))


def _dense_body(x_ref, o0_ref, o1_ref, w_ref, b_ref, g_ref, bt_ref,
                xn_ref, h_ref):
    o0 = o0_ref[0]
    o1 = o1_ref[0]
    a0 = o0[:, :HH] / (o0[:, HH:] + 1e-16)
    a1 = o1[:, :HH] / (o1[:, HH:] + 1e-16)
    xn = (x_ref[...]
          + jnp.dot(a0, w_ref[0], preferred_element_type=F32)
          + jnp.dot(a1, w_ref[1], preferred_element_type=F32)
          + b_ref[...])
    xn_ref[...] = xn
    mu = jnp.mean(xn, axis=1, keepdims=True)
    var = jnp.mean((xn - mu) ** 2, axis=1, keepdims=True)
    h = (xn - mu) * lax.rsqrt(var + 1e-5) * g_ref[...] + bt_ref[...]
    h = jnp.maximum(h, 0.0)
    h_ref[...] = jnp.stack([h[:, :HH], h[:, HH:]], axis=0)


def _dense_step(x, accum, w2, b, g_next, b_next):
    n = x.shape[0]
    grid = n // ROWBLK
    return pl.pallas_call(
        _dense_body,
        grid=(grid,),
        in_specs=[
            pl.BlockSpec((ROWBLK, H), lambda i: (i, 0)),
            pl.BlockSpec((1, ROWBLK, H), lambda i: (0, i, 0)),
            pl.BlockSpec((1, ROWBLK, H), lambda i: (1, i, 0)),
            pl.BlockSpec((2, HH, H), lambda i: (0, 0, 0)),
            pl.BlockSpec((1, H), lambda i: (0, 0)),
            pl.BlockSpec((1, H), lambda i: (0, 0)),
            pl.BlockSpec((1, H), lambda i: (0, 0)),
        ],
        out_specs=[
            pl.BlockSpec((ROWBLK, H), lambda i: (i, 0)),
            pl.BlockSpec((2, ROWBLK, HH), lambda i: (0, i, 0)),
        ],
        out_shape=[
            jax.ShapeDtypeStruct((n, H), F32),
            jax.ShapeDtypeStruct((2, n, HH), F32),
        ],
    )(x, accum, accum, w2, b.reshape(1, H), g_next.reshape(1, H),
      b_next.reshape(1, H))


def _head_body(p1_ref, p2_ref, w1_ref, b1_ref, g1_ref, t1_ref,
               w2_ref, b2_ref, g2_ref, t2_ref, w3_ref, b3_ref, o_ref):
    def mean_pool(p):
        acc = p[0] + p[1]
        s = acc[:, :H]
        cnt = acc[:, H:]
        return s / jnp.maximum(cnt, 1.0)

    m1 = mean_pool(p1_ref[...])
    m2 = mean_pool(p2_ref[...])
    hh = (jnp.dot(m1, w1_ref[0], preferred_element_type=F32)
          + jnp.dot(m2, w1_ref[1], preferred_element_type=F32)
          + b1_ref[...])
    hh = jnp.maximum(hh * g1_ref[...] + t1_ref[...], 0.0)
    h2 = jnp.dot(hh, w2_ref[...], preferred_element_type=F32) + b2_ref[...]
    h2 = jnp.maximum(h2 * g2_ref[...] + t2_ref[...], 0.0)
    o = jnp.dot(h2, w3_ref[...], preferred_element_type=F32) + b3_ref[...]
    o_ref[...] = jax.nn.sigmoid(o)


def _head(p1, p2, fc1_W, fc1_b, bn1_g, bn1_b, fc2_W, fc2_b, bn2_g, bn2_b,
          out_W, out_b):
    full = lambda *s: pl.BlockSpec(s, lambda: tuple(0 for _ in s))
    return pl.pallas_call(
        _head_body,
        in_specs=[
            full(2, NGRAPH, 2 * H),
            full(2, NGRAPH, 2 * H),
            full(2, H, H),
            full(1, H), full(1, H), full(1, H),
            full(H, 32), full(1, 32), full(1, 32), full(1, 32),
            full(32, 1), full(1, 1),
        ],
        out_specs=full(NGRAPH, 1),
        out_shape=jax.ShapeDtypeStruct((NGRAPH, 1), F32),
    )(p1, p2, fc1_W.reshape(2, H, H), fc1_b.reshape(1, H),
      bn1_g.reshape(1, H), bn1_b.reshape(1, H), fc2_W, fc2_b.reshape(1, 32),
      bn2_g.reshape(1, 32), bn2_b.reshape(1, 32), out_W, out_b.reshape(1, 1))


EDGE_BLK = 80
NTILE = 16
NPAD = 10112


def _edge_sc_body(h_ref, e_ref, src_ref, dst_ref, out_ref,
                  accum, srcv0, dstv0, srcv1, dstv1,
                  hrows0, erows0, hrows1, erows1, vals,
                  sem_h, sem_e, sem_i):
    c = lax.axis_index("c")
    s = lax.axis_index("s")
    n = h_ref.shape[0] // 2
    e_total = src_ref.shape[0]
    per_tile = e_total // NTILE
    nblk = per_tile // EDGE_BLK
    rows_per_tile = NPAD // NTILE
    tile_e0 = s * per_tile
    cn = c * n

    def _zb(r, _):
        for k in range(H // 16):
            vals[r, pl.ds(k * 16, 16)] = jnp.zeros((16,), F32)
        return 0
    lax.fori_loop(0, EDGE_BLK, _zb, 0)
    base = s * rows_per_tile
    nfull = rows_per_tile // EDGE_BLK
    rem = rows_per_tile - nfull * EDGE_BLK
    for j in range(nfull):
        pltpu.sync_copy(vals, accum.at[pl.ds(base + j * EDGE_BLK, EDGE_BLK)])
    if rem:
        pltpu.sync_copy(vals.at[pl.ds(0, rem)],
                        accum.at[pl.ds(base + nfull * EDGE_BLK, rem)])
    plsc.subcore_barrier()

    def _issue_idx(k, sv, dv):
        off = tile_e0 + k * EDGE_BLK
        pltpu.async_copy(src_ref.at[pl.ds(off, EDGE_BLK)], sv, sem_i)
        pltpu.async_copy(dst_ref.at[pl.ds(off, EDGE_BLK)], dv, sem_i)

    def _wait_idx(sv, dv):
        pltpu.make_async_copy(src_ref.at[pl.ds(0, EDGE_BLK)], sv, sem_i).wait()
        pltpu.make_async_copy(dst_ref.at[pl.ds(0, EDGE_BLK)], dv, sem_i).wait()

    def _adjust(sv):
        def _a(i, _):
            sv[pl.ds(i * 16, 16)] = sv[pl.ds(i * 16, 16)] + cn
            return 0
        lax.fori_loop(0, EDGE_BLK // 16, _a, 0)

    def _issue_data(k, sv, hb, eb):
        pltpu.async_copy(h_ref.at[sv], hb, sem_h)
        off = c * e_total + tile_e0 + k * EDGE_BLK
        pltpu.async_copy(e_ref.at[pl.ds(off, EDGE_BLK)], eb, sem_e)

    def _wait_data(sv, hb, eb):
        pltpu.make_async_copy(h_ref.at[sv], hb, sem_h).wait()
        pltpu.make_async_copy(e_ref.at[pl.ds(0, EDGE_BLK)], eb, sem_e).wait()

    def _compute(hb, eb):
        def _row(b, _):
            for k2 in range(HH // 16):
                hv = hb[b, pl.ds(k2 * 16, 16)]
                ev = eb[b, pl.ds(k2 * 16, 16)]
                m = jnp.maximum(hv + ev, 0.0)
                x = jnp.exp(m)
                vals[b, pl.ds(k2 * 16, 16)] = x * m
                vals[b, pl.ds(HH + k2 * 16, 16)] = x
            return 0
        lax.fori_loop(0, EDGE_BLK, _row, 0)

    pltpu.sync_copy(src_ref.at[pl.ds(tile_e0, EDGE_BLK)], srcv0)
    pltpu.sync_copy(dst_ref.at[pl.ds(tile_e0, EDGE_BLK)], dstv0)
    _adjust(srcv0)
    _issue_data(0, srcv0, hrows0, erows0)
    _issue_idx(1, srcv1, dstv1)

    bufs = ((srcv0, dstv0, hrows0, erows0), (srcv1, dstv1, hrows1, erows1))

    def _slot(k, p):
        cur = bufs[p]
        nxt = bufs[p ^ 1]

        @pl.when(k < nblk)
        def _():
            @pl.when(k + 1 < nblk)
            def _():
                _wait_idx(nxt[0], nxt[1])
                _adjust(nxt[0])
                _issue_data(k + 1, nxt[0], nxt[2], nxt[3])

            _wait_data(cur[0], cur[2], cur[3])
            _compute(cur[2], cur[3])
            pltpu.sync_copy(vals, accum.at[cur[1]], add=True)

            @pl.when(k + 2 < nblk)
            def _():
                _issue_idx(k + 2, cur[0], cur[1])

    def _pair(i, _):
        _slot(2 * i, 0)
        _slot(2 * i + 1, 1)
        return 0
    lax.fori_loop(0, (nblk + 1) // 2, _pair, 0)

    plsc.subcore_barrier()
    pltpu.sync_copy(accum.at[pl.ds(s * rows_per_tile, rows_per_tile)],
                    out_ref.at[pl.ds(c * NPAD + s * rows_per_tile,
                                     rows_per_tile)])


def _edge_pass(h_split, e2, src, dst, n):
    h2 = h_split.reshape(2 * n, HH)
    mesh = plsc.VectorSubcoreMesh(core_axis_name="c", subcore_axis_name="s")
    out = pl.kernel(
        _edge_sc_body,
        mesh=mesh,
        compiler_params=pltpu.CompilerParams(use_tc_tiling_on_sc=False),
        out_type=jax.ShapeDtypeStruct((2 * NPAD, H), F32),
        scratch_types=[
            pltpu.VMEM_SHARED((NPAD, H), F32),
            pltpu.VMEM((EDGE_BLK,), jnp.int32),
            pltpu.VMEM((EDGE_BLK,), jnp.int32),
            pltpu.VMEM((EDGE_BLK,), jnp.int32),
            pltpu.VMEM((EDGE_BLK,), jnp.int32),
            pltpu.VMEM((EDGE_BLK, HH), F32),
            pltpu.VMEM((EDGE_BLK, HH), F32),
            pltpu.VMEM((EDGE_BLK, HH), F32),
            pltpu.VMEM((EDGE_BLK, HH), F32),
            pltpu.VMEM((EDGE_BLK, H), F32),
            pltpu.SemaphoreType.DMA,
            pltpu.SemaphoreType.DMA,
            pltpu.SemaphoreType.DMA,
        ],
    )(h2, e2, src, dst)
    return out.reshape(2, NPAD, H)


def _pool(x, batch):
    s = jax.ops.segment_sum(x, batch, num_segments=NGRAPH)
    cnt = jax.ops.segment_sum(jnp.ones((x.shape[0],), F32), batch,
                              num_segments=NGRAPH)
    cntb = jnp.broadcast_to(cnt[:, None], (NGRAPH, H))
    part = jnp.concatenate([s, cntb], axis=1)
    return jnp.stack([part, jnp.zeros_like(part)])


def kernel(entry1_x, entry1_edge_index, entry1_edge_attr, entry1_batch,
           entry2_x, entry2_edge_index, entry2_edge_attr, entry2_batch,
           node_W, node_b, edge_W, edge_b, ln_g, ln_b, mlp_W, mlp_b,
           fc1_W, fc1_b, bn1_g, bn1_b, fc2_W, fc2_b, bn2_g, bn2_b,
           out_W, out_b):
    num_layers = mlp_W.shape[0]
    n = entry1_x.shape[0]
    e_num = entry1_edge_index.shape[1]
    src1, dst1 = entry1_edge_index[0], entry1_edge_index[1]
    src2, dst2 = entry2_edge_index[0], entry2_edge_index[1]
    e1 = _edge_proj(entry1_edge_attr, edge_W, edge_b).reshape(2 * e_num, HH)
    e2 = _edge_proj(entry2_edge_attr, edge_W, edge_b).reshape(2 * e_num, HH)
    x1, h1 = _proj(entry1_x, node_W, node_b, ln_g[0], ln_b[0])
    x2, h2 = _proj(entry2_x, node_W, node_b, ln_g[0], ln_b[0])
    for l in range(num_layers):
        a1 = _edge_pass(h1, e1, src1, dst1, n)
        a2 = _edge_pass(h2, e2, src2, dst2, n)
        nl = min(l + 1, num_layers - 1)
        w2 = mlp_W[l].reshape(2, HH, H)
        x1, h1 = _dense_step(x1, a1, w2, mlp_b[l], ln_g[nl], ln_b[nl])
        x2, h2 = _dense_step(x2, a2, w2, mlp_b[l], ln_g[nl], ln_b[nl])
    p1 = _pool(x1, entry1_batch)
    p2 = _pool(x2, entry2_batch)
    return _head(p1, p2, fc1_W, fc1_b, bn1_g, bn1_b, fc2_W, fc2_b,
                 bn2_g, bn2_b, out_W, out_b)


# speedup vs baseline: 1.3574x; 1.0861x over previous
import functools

import jax
import jax.numpy as jnp
from jax import lax
from jax.experimental import pallas as pl
from jax.experimental.pallas import tpu as pltpu
from jax.experimental.pallas import tpu_sc as plsc

F32 = jnp.float32
NGRAPH = 256
H = 128
HH = 64
ROWBLK = 2000


def _proj_body(x_ref, w_ref, b_ref, g_ref, bt_ref, x0_ref, h_ref):
    x0 = jnp.dot(x_ref[...], w_ref[...], preferred_element_type=F32) + b_ref[...]
    x0_ref[...] = x0
    mu = jnp.mean(x0, axis=1, keepdims=True)
    var = jnp.mean((x0 - mu) ** 2, axis=1, keepdims=True)
    h = (x0 - mu) * lax.rsqrt(var + 1e-5) * g_ref[...] + bt_ref[...]
    h = jnp.maximum(h, 0.0)
    h_ref[...] = jnp.stack([h[:, :HH], h[:, HH:]], axis=0)


def _proj(x, node_W, node_b, g0, b0):
    n = x.shape[0]
    grid = n // ROWBLK
    return pl.pallas_call(
        _proj_body,
        grid=(grid,),
        in_specs=[
            pl.BlockSpec((ROWBLK, H), lambda i: (i, 0)),
            pl.BlockSpec((H, H), lambda i: (0, 0)),
            pl.BlockSpec((1, H), lambda i: (0, 0)),
            pl.BlockSpec((1, H), lambda i: (0, 0)),
            pl.BlockSpec((1, H), lambda i: (0, 0)),
        ],
        out_specs=[
            pl.BlockSpec((ROWBLK, H), lambda i: (i, 0)),
            pl.BlockSpec((2, ROWBLK, HH), lambda i: (0, i, 0)),
        ],
        out_shape=[
            jax.ShapeDtypeStruct((n, H), F32),
            jax.ShapeDtypeStruct((2, n, HH), F32),
        ],
    )(x, node_W, node_b.reshape(1, H), g0.reshape(1, H), b0.reshape(1, H))


def _edge_proj_body(a_ref, w_ref, b_ref, e_ref):
    e = jnp.dot(a_ref[...], w_ref[...], preferred_element_type=F32) + b_ref[...]
    e_ref[...] = jnp.stack([e[:, :HH], e[:, HH:]], axis=0)


def _edge_proj(edge_attr, edge_W, edge_b):
    e_num, de = edge_attr.shape
    grid = e_num // ROWBLK
    return pl.pallas_call(
        _edge_proj_body,
        grid=(grid,),
        in_specs=[
            pl.BlockSpec((ROWBLK, de), lambda i: (i, 0)),
            pl.BlockSpec((de, H), lambda i: (0, 0)),
            pl.BlockSpec((1, H), lambda i: (0, 0)),
        ],
        out_specs=pl.BlockSpec((2, ROWBLK, HH), lambda i: (0, i, 0)),
        out_shape=jax.ShapeDtypeStruct((2, e_num, HH), F32),
    )(edge_attr, edge_W, edge_b.reshape(1, H))


def _dense_body(x_ref, o0_ref, o1_ref, w_ref, b_ref, g_ref, bt_ref,
                xn_ref, h_ref):
    o0 = o0_ref[0]
    o1 = o1_ref[0]
    a0 = o0[:, :HH] / (o0[:, HH:] + 1e-16)
    a1 = o1[:, :HH] / (o1[:, HH:] + 1e-16)
    xn = (x_ref[...]
          + jnp.dot(a0, w_ref[0], preferred_element_type=F32)
          + jnp.dot(a1, w_ref[1], preferred_element_type=F32)
          + b_ref[...])
    xn_ref[...] = xn
    mu = jnp.mean(xn, axis=1, keepdims=True)
    var = jnp.mean((xn - mu) ** 2, axis=1, keepdims=True)
    h = (xn - mu) * lax.rsqrt(var + 1e-5) * g_ref[...] + bt_ref[...]
    h = jnp.maximum(h, 0.0)
    h_ref[...] = jnp.stack([h[:, :HH], h[:, HH:]], axis=0)


def _dense_step(x, accum, w2, b, g_next, b_next):
    n = x.shape[0]
    grid = n // ROWBLK
    return pl.pallas_call(
        _dense_body,
        grid=(grid,),
        in_specs=[
            pl.BlockSpec((ROWBLK, H), lambda i: (i, 0)),
            pl.BlockSpec((1, ROWBLK, H), lambda i: (0, i, 0)),
            pl.BlockSpec((1, ROWBLK, H), lambda i: (1, i, 0)),
            pl.BlockSpec((2, HH, H), lambda i: (0, 0, 0)),
            pl.BlockSpec((1, H), lambda i: (0, 0)),
            pl.BlockSpec((1, H), lambda i: (0, 0)),
            pl.BlockSpec((1, H), lambda i: (0, 0)),
        ],
        out_specs=[
            pl.BlockSpec((ROWBLK, H), lambda i: (i, 0)),
            pl.BlockSpec((2, ROWBLK, HH), lambda i: (0, i, 0)),
        ],
        out_shape=[
            jax.ShapeDtypeStruct((n, H), F32),
            jax.ShapeDtypeStruct((2, n, HH), F32),
        ],
    )(x, accum, accum, w2, b.reshape(1, H), g_next.reshape(1, H),
      b_next.reshape(1, H))


def _head_body(p1_ref, p2_ref, w1_ref, b1_ref, g1_ref, t1_ref,
               w2_ref, b2_ref, g2_ref, t2_ref, w3_ref, b3_ref, o_ref):
    def mean_pool(p):
        acc = p[0] + p[1]
        s = acc[:, :H]
        cnt = acc[:, H:]
        return s / jnp.maximum(cnt, 1.0)

    m1 = mean_pool(p1_ref[...])
    m2 = mean_pool(p2_ref[...])
    hh = (jnp.dot(m1, w1_ref[0], preferred_element_type=F32)
          + jnp.dot(m2, w1_ref[1], preferred_element_type=F32)
          + b1_ref[...])
    hh = jnp.maximum(hh * g1_ref[...] + t1_ref[...], 0.0)
    h2 = jnp.dot(hh, w2_ref[...], preferred_element_type=F32) + b2_ref[...]
    h2 = jnp.maximum(h2 * g2_ref[...] + t2_ref[...], 0.0)
    o = jnp.dot(h2, w3_ref[...], preferred_element_type=F32) + b3_ref[...]
    o_ref[...] = jax.nn.sigmoid(o)


def _head(p1, p2, fc1_W, fc1_b, bn1_g, bn1_b, fc2_W, fc2_b, bn2_g, bn2_b,
          out_W, out_b):
    full = lambda *s: pl.BlockSpec(s, lambda: tuple(0 for _ in s))
    return pl.pallas_call(
        _head_body,
        in_specs=[
            full(2, NGRAPH, 2 * H),
            full(2, NGRAPH, 2 * H),
            full(2, H, H),
            full(1, H), full(1, H), full(1, H),
            full(H, 32), full(1, 32), full(1, 32), full(1, 32),
            full(32, 1), full(1, 1),
        ],
        out_specs=full(NGRAPH, 1),
        out_shape=jax.ShapeDtypeStruct((NGRAPH, 1), F32),
    )(p1, p2, fc1_W.reshape(2, H, H), fc1_b.reshape(1, H),
      bn1_g.reshape(1, H), bn1_b.reshape(1, H), fc2_W, fc2_b.reshape(1, 32),
      bn2_g.reshape(1, 32), bn2_b.reshape(1, 32), out_W, out_b.reshape(1, 1))


EDGE_BLK = 80
NTILE = 16
NPAD = 10112


def _edge_sc_body(h_ref, e_ref, src_ref, dst_ref, out_ref,
                  accum, srcv0, srcv1, dstv0, dstv1, dstv2, dstv3,
                  hrows0, erows0, hrows1, erows1, vals0, vals1,
                  sem_h, sem_e, sem_i, sem_s):
    c = lax.axis_index("c")
    s = lax.axis_index("s")
    n = h_ref.shape[0] // 2
    e_total = src_ref.shape[0]
    per_tile = e_total // NTILE
    nblk = per_tile // EDGE_BLK
    rows_per_tile = NPAD // NTILE
    tile_e0 = s * per_tile
    cn = c * n

    def _zb(r, _):
        for k in range(H // 16):
            vals0[r, pl.ds(k * 16, 16)] = jnp.zeros((16,), F32)
        return 0
    lax.fori_loop(0, EDGE_BLK, _zb, 0)
    base = s * rows_per_tile
    nfull = rows_per_tile // EDGE_BLK
    rem = rows_per_tile - nfull * EDGE_BLK
    for j in range(nfull):
        pltpu.sync_copy(vals0, accum.at[pl.ds(base + j * EDGE_BLK, EDGE_BLK)])
    if rem:
        pltpu.sync_copy(vals0.at[pl.ds(0, rem)],
                        accum.at[pl.ds(base + nfull * EDGE_BLK, rem)])
    plsc.subcore_barrier()

    def _issue_idx(k, sv, dv):
        off = tile_e0 + k * EDGE_BLK
        pltpu.async_copy(src_ref.at[pl.ds(off, EDGE_BLK)], sv, sem_i)
        pltpu.async_copy(dst_ref.at[pl.ds(off, EDGE_BLK)], dv, sem_i)

    def _wait_idx(sv, dv):
        pltpu.make_async_copy(src_ref.at[pl.ds(0, EDGE_BLK)], sv, sem_i).wait()
        pltpu.make_async_copy(dst_ref.at[pl.ds(0, EDGE_BLK)], dv, sem_i).wait()

    def _adjust(sv):
        for i in range(EDGE_BLK // 16):
            sv[pl.ds(i * 16, 16)] = sv[pl.ds(i * 16, 16)] + cn

    def _issue_data(k, sv, hb, eb):
        pltpu.async_copy(h_ref.at[sv], hb, sem_h)
        off = c * e_total + tile_e0 + k * EDGE_BLK
        pltpu.async_copy(e_ref.at[pl.ds(off, EDGE_BLK)], eb, sem_e)

    def _wait_data(sv, hb, eb):
        pltpu.make_async_copy(h_ref.at[sv], hb, sem_h).wait()
        pltpu.make_async_copy(e_ref.at[pl.ds(0, EDGE_BLK)], eb, sem_e).wait()

    def _compute(hb, eb, vb):
        def _row(i, _):
            for u in range(2):
                for k2 in range(HH // 16):
                    hv = hb[2 * i + u, pl.ds(k2 * 16, 16)]
                    ev = eb[2 * i + u, pl.ds(k2 * 16, 16)]
                    m = jnp.maximum(hv + ev, 0.0)
                    x = jnp.exp(m)
                    vb[2 * i + u, pl.ds(k2 * 16, 16)] = x * m
                    vb[2 * i + u, pl.ds(HH + k2 * 16, 16)] = x
            return 0
        lax.fori_loop(0, EDGE_BLK // 2, _row, 0)

    def _wait_scatter(vb):
        pltpu.make_async_copy(vb, accum.at[dstv0], sem_s).wait()

    pltpu.sync_copy(src_ref.at[pl.ds(tile_e0, EDGE_BLK)], srcv0)
    pltpu.sync_copy(dst_ref.at[pl.ds(tile_e0, EDGE_BLK)], dstv0)
    _adjust(srcv0)
    _issue_data(0, srcv0, hrows0, erows0)
    _issue_idx(1, srcv1, dstv1)

    srcs = (srcv0, srcv1)
    dsts = (dstv0, dstv1, dstv2, dstv3)
    hrs = (hrows0, hrows1)
    ers = (erows0, erows1)
    vls = (vals0, vals1)

    def _slot(k, p, d):
        @pl.when(k < nblk)
        def _():
            @pl.when(k + 1 < nblk)
            def _():
                _wait_idx(srcs[p ^ 1], dsts[(d + 1) % 4])
                _adjust(srcs[p ^ 1])
                _issue_data(k + 1, srcs[p ^ 1], hrs[p ^ 1], ers[p ^ 1])

            _wait_data(srcs[p], hrs[p], ers[p])

            @pl.when(k >= 2)
            def _():
                _wait_scatter(vls[p])

            _compute(hrs[p], ers[p], vls[p])
            pltpu.async_copy(vls[p], accum.at[dsts[d]], sem_s, add=True)

            @pl.when(k + 2 < nblk)
            def _():
                _issue_idx(k + 2, srcs[p], dsts[(d + 2) % 4])

    def _quad(i, _):
        for u in range(4):
            _slot(4 * i + u, u % 2, u)
        return 0
    lax.fori_loop(0, (nblk + 3) // 4, _quad, 0)

    _wait_scatter(vls[0])
    _wait_scatter(vls[1])
    plsc.subcore_barrier()
    pltpu.sync_copy(accum.at[pl.ds(s * rows_per_tile, rows_per_tile)],
                    out_ref.at[pl.ds(c * NPAD + s * rows_per_tile,
                                     rows_per_tile)])


def _edge_pass(h_split, e2, src, dst, n):
    h2 = h_split.reshape(2 * n, HH)
    mesh = plsc.VectorSubcoreMesh(core_axis_name="c", subcore_axis_name="s")
    out = pl.kernel(
        _edge_sc_body,
        mesh=mesh,
        compiler_params=pltpu.CompilerParams(use_tc_tiling_on_sc=False),
        out_type=jax.ShapeDtypeStruct((2 * NPAD, H), F32),
        scratch_types=[
            pltpu.VMEM_SHARED((NPAD, H), F32),
            pltpu.VMEM((EDGE_BLK,), jnp.int32),
            pltpu.VMEM((EDGE_BLK,), jnp.int32),
            pltpu.VMEM((EDGE_BLK,), jnp.int32),
            pltpu.VMEM((EDGE_BLK,), jnp.int32),
            pltpu.VMEM((EDGE_BLK,), jnp.int32),
            pltpu.VMEM((EDGE_BLK,), jnp.int32),
            pltpu.VMEM((EDGE_BLK, HH), F32),
            pltpu.VMEM((EDGE_BLK, HH), F32),
            pltpu.VMEM((EDGE_BLK, HH), F32),
            pltpu.VMEM((EDGE_BLK, HH), F32),
            pltpu.VMEM((EDGE_BLK, H), F32),
            pltpu.VMEM((EDGE_BLK, H), F32),
            pltpu.SemaphoreType.DMA,
            pltpu.SemaphoreType.DMA,
            pltpu.SemaphoreType.DMA,
            pltpu.SemaphoreType.DMA,
        ],
    )(h2, e2, src, dst)
    return out.reshape(2, NPAD, H)


def _pool(x, batch):
    s = jax.ops.segment_sum(x, batch, num_segments=NGRAPH)
    cnt = jax.ops.segment_sum(jnp.ones((x.shape[0],), F32), batch,
                              num_segments=NGRAPH)
    cntb = jnp.broadcast_to(cnt[:, None], (NGRAPH, H))
    part = jnp.concatenate([s, cntb], axis=1)
    return jnp.stack([part, jnp.zeros_like(part)])


def kernel(entry1_x, entry1_edge_index, entry1_edge_attr, entry1_batch,
           entry2_x, entry2_edge_index, entry2_edge_attr, entry2_batch,
           node_W, node_b, edge_W, edge_b, ln_g, ln_b, mlp_W, mlp_b,
           fc1_W, fc1_b, bn1_g, bn1_b, fc2_W, fc2_b, bn2_g, bn2_b,
           out_W, out_b):
    num_layers = mlp_W.shape[0]
    n = entry1_x.shape[0]
    e_num = entry1_edge_index.shape[1]
    src1, dst1 = entry1_edge_index[0], entry1_edge_index[1]
    src2, dst2 = entry2_edge_index[0], entry2_edge_index[1]
    e1 = _edge_proj(entry1_edge_attr, edge_W, edge_b).reshape(2 * e_num, HH)
    e2 = _edge_proj(entry2_edge_attr, edge_W, edge_b).reshape(2 * e_num, HH)
    x1, h1 = _proj(entry1_x, node_W, node_b, ln_g[0], ln_b[0])
    x2, h2 = _proj(entry2_x, node_W, node_b, ln_g[0], ln_b[0])
    for l in range(num_layers):
        a1 = _edge_pass(h1, e1, src1, dst1, n)
        a2 = _edge_pass(h2, e2, src2, dst2, n)
        nl = min(l + 1, num_layers - 1)
        w2 = mlp_W[l].reshape(2, HH, H)
        x1, h1 = _dense_step(x1, a1, w2, mlp_b[l], ln_g[nl], ln_b[nl])
        x2, h2 = _dense_step(x2, a2, w2, mlp_b[l], ln_g[nl], ln_b[nl])
    p1 = _pool(x1, entry1_batch)
    p2 = _pool(x2, entry2_batch)
    return _head(p1, p2, fc1_W, fc1_b, bn1_g, bn1_b, fc2_W, fc2_b,
                 bn2_g, bn2_b, out_W, out_b)


# speedup vs baseline: 1.3597x; 1.0017x over previous
import functools

import jax
import jax.numpy as jnp
from jax import lax
from jax.experimental import pallas as pl
from jax.experimental.pallas import tpu as pltpu
from jax.experimental.pallas import tpu_sc as plsc

F32 = jnp.float32
NGRAPH = 256
H = 128
HH = 64
ROWBLK = 2000


def _proj_body(x_ref, w_ref, b_ref, g_ref, bt_ref, x0_ref, h_ref):
    x0 = jnp.dot(x_ref[...], w_ref[...], preferred_element_type=F32) + b_ref[...]
    x0_ref[...] = x0
    mu = jnp.mean(x0, axis=1, keepdims=True)
    var = jnp.mean((x0 - mu) ** 2, axis=1, keepdims=True)
    h = (x0 - mu) * lax.rsqrt(var + 1e-5) * g_ref[...] + bt_ref[...]
    h = jnp.maximum(h, 0.0)
    h_ref[...] = jnp.stack([h[:, :HH], h[:, HH:]], axis=0)


def _proj(x, node_W, node_b, g0, b0):
    n = x.shape[0]
    grid = n // ROWBLK
    return pl.pallas_call(
        _proj_body,
        grid=(grid,),
        in_specs=[
            pl.BlockSpec((ROWBLK, H), lambda i: (i, 0)),
            pl.BlockSpec((H, H), lambda i: (0, 0)),
            pl.BlockSpec((1, H), lambda i: (0, 0)),
            pl.BlockSpec((1, H), lambda i: (0, 0)),
            pl.BlockSpec((1, H), lambda i: (0, 0)),
        ],
        out_specs=[
            pl.BlockSpec((ROWBLK, H), lambda i: (i, 0)),
            pl.BlockSpec((2, ROWBLK, HH), lambda i: (0, i, 0)),
        ],
        out_shape=[
            jax.ShapeDtypeStruct((n, H), F32),
            jax.ShapeDtypeStruct((2, n, HH), F32),
        ],
    )(x, node_W, node_b.reshape(1, H), g0.reshape(1, H), b0.reshape(1, H))


def _edge_proj_body(a_ref, w_ref, b_ref, e_ref):
    e = jnp.dot(a_ref[...], w_ref[...], preferred_element_type=F32) + b_ref[...]
    e_ref[...] = jnp.stack([e[:, :HH], e[:, HH:]], axis=0)


def _edge_proj(edge_attr, edge_W, edge_b):
    e_num, de = edge_attr.shape
    grid = e_num // ROWBLK
    return pl.pallas_call(
        _edge_proj_body,
        grid=(grid,),
        in_specs=[
            pl.BlockSpec((ROWBLK, de), lambda i: (i, 0)),
            pl.BlockSpec((de, H), lambda i: (0, 0)),
            pl.BlockSpec((1, H), lambda i: (0, 0)),
        ],
        out_specs=pl.BlockSpec((2, ROWBLK, HH), lambda i: (0, i, 0)),
        out_shape=jax.ShapeDtypeStruct((2, e_num, HH), F32),
    )(edge_attr, edge_W, edge_b.reshape(1, H))


def _dense_body(x_ref, o0_ref, o1_ref, w_ref, b_ref, g_ref, bt_ref,
                xn_ref, h_ref):
    o0 = o0_ref[0]
    o1 = o1_ref[0]
    a0 = o0[:, :HH] / (o0[:, HH:] + 1e-16)
    a1 = o1[:, :HH] / (o1[:, HH:] + 1e-16)
    xn = (x_ref[...]
          + jnp.dot(a0, w_ref[0], preferred_element_type=F32)
          + jnp.dot(a1, w_ref[1], preferred_element_type=F32)
          + b_ref[...])
    xn_ref[...] = xn
    mu = jnp.mean(xn, axis=1, keepdims=True)
    var = jnp.mean((xn - mu) ** 2, axis=1, keepdims=True)
    h = (xn - mu) * lax.rsqrt(var + 1e-5) * g_ref[...] + bt_ref[...]
    h = jnp.maximum(h, 0.0)
    h_ref[...] = jnp.stack([h[:, :HH], h[:, HH:]], axis=0)


def _dense_step(x, accum, w2, b, g_next, b_next):
    n = x.shape[0]
    grid = n // ROWBLK
    return pl.pallas_call(
        _dense_body,
        grid=(grid,),
        in_specs=[
            pl.BlockSpec((ROWBLK, H), lambda i: (i, 0)),
            pl.BlockSpec((1, ROWBLK, H), lambda i: (0, i, 0)),
            pl.BlockSpec((1, ROWBLK, H), lambda i: (1, i, 0)),
            pl.BlockSpec((2, HH, H), lambda i: (0, 0, 0)),
            pl.BlockSpec((1, H), lambda i: (0, 0)),
            pl.BlockSpec((1, H), lambda i: (0, 0)),
            pl.BlockSpec((1, H), lambda i: (0, 0)),
        ],
        out_specs=[
            pl.BlockSpec((ROWBLK, H), lambda i: (i, 0)),
            pl.BlockSpec((2, ROWBLK, HH), lambda i: (0, i, 0)),
        ],
        out_shape=[
            jax.ShapeDtypeStruct((n, H), F32),
            jax.ShapeDtypeStruct((2, n, HH), F32),
        ],
    )(x, accum, accum, w2, b.reshape(1, H), g_next.reshape(1, H),
      b_next.reshape(1, H))


def _head_body(p1_ref, p2_ref, w1_ref, b1_ref, g1_ref, t1_ref,
               w2_ref, b2_ref, g2_ref, t2_ref, w3_ref, b3_ref, o_ref):
    def mean_pool(p):
        acc = p[0] + p[1]
        s = acc[:, :H]
        cnt = acc[:, H:]
        return s / jnp.maximum(cnt, 1.0)

    m1 = mean_pool(p1_ref[...])
    m2 = mean_pool(p2_ref[...])
    hh = (jnp.dot(m1, w1_ref[0], preferred_element_type=F32)
          + jnp.dot(m2, w1_ref[1], preferred_element_type=F32)
          + b1_ref[...])
    hh = jnp.maximum(hh * g1_ref[...] + t1_ref[...], 0.0)
    h2 = jnp.dot(hh, w2_ref[...], preferred_element_type=F32) + b2_ref[...]
    h2 = jnp.maximum(h2 * g2_ref[...] + t2_ref[...], 0.0)
    o = jnp.dot(h2, w3_ref[...], preferred_element_type=F32) + b3_ref[...]
    o_ref[...] = jax.nn.sigmoid(o)


def _head(p1, p2, fc1_W, fc1_b, bn1_g, bn1_b, fc2_W, fc2_b, bn2_g, bn2_b,
          out_W, out_b):
    full = lambda *s: pl.BlockSpec(s, lambda: tuple(0 for _ in s))
    return pl.pallas_call(
        _head_body,
        in_specs=[
            full(2, NGRAPH, 2 * H),
            full(2, NGRAPH, 2 * H),
            full(2, H, H),
            full(1, H), full(1, H), full(1, H),
            full(H, 32), full(1, 32), full(1, 32), full(1, 32),
            full(32, 1), full(1, 1),
        ],
        out_specs=full(NGRAPH, 1),
        out_shape=jax.ShapeDtypeStruct((NGRAPH, 1), F32),
    )(p1, p2, fc1_W.reshape(2, H, H), fc1_b.reshape(1, H),
      bn1_g.reshape(1, H), bn1_b.reshape(1, H), fc2_W, fc2_b.reshape(1, 32),
      bn2_g.reshape(1, 32), bn2_b.reshape(1, 32), out_W, out_b.reshape(1, 1))


EDGE_BLK = 80
NTILE = 16
NPAD = 10112


def _edge_sc_body(h_ref, e_ref, src_ref, dst_ref, out_ref,
                  accum, srcv0, srcv1, dstv0, dstv1, dstv2, dstv3,
                  hrows0, erows0, hrows1, erows1, vals0, vals1,
                  sem_h, sem_e, sem_i, sem_s):
    c = lax.axis_index("c")
    s = lax.axis_index("s")
    n = h_ref.shape[0] // 2
    e_total = src_ref.shape[0]
    per_tile = e_total // NTILE
    nblk = per_tile // EDGE_BLK
    rows_per_tile = NPAD // NTILE
    tile_e0 = s * per_tile
    cn = c * n

    def _zb(r, _):
        for k in range(H // 16):
            vals0[r, pl.ds(k * 16, 16)] = jnp.zeros((16,), F32)
        return 0
    lax.fori_loop(0, EDGE_BLK, _zb, 0)
    base = s * rows_per_tile
    nfull = rows_per_tile // EDGE_BLK
    rem = rows_per_tile - nfull * EDGE_BLK
    for j in range(nfull):
        pltpu.sync_copy(vals0, accum.at[pl.ds(base + j * EDGE_BLK, EDGE_BLK)])
    if rem:
        pltpu.sync_copy(vals0.at[pl.ds(0, rem)],
                        accum.at[pl.ds(base + nfull * EDGE_BLK, rem)])
    plsc.subcore_barrier()

    def _issue_idx(k, sv, dv):
        off = tile_e0 + k * EDGE_BLK
        pltpu.async_copy(src_ref.at[pl.ds(off, EDGE_BLK)], sv, sem_i)
        pltpu.async_copy(dst_ref.at[pl.ds(off, EDGE_BLK)], dv, sem_i)

    def _wait_idx(sv, dv):
        pltpu.make_async_copy(src_ref.at[pl.ds(0, EDGE_BLK)], sv, sem_i).wait()
        pltpu.make_async_copy(dst_ref.at[pl.ds(0, EDGE_BLK)], dv, sem_i).wait()

    def _adjust(sv):
        for i in range(EDGE_BLK // 16):
            sv[pl.ds(i * 16, 16)] = sv[pl.ds(i * 16, 16)] + cn

    def _issue_data(k, sv, hb, eb):
        pltpu.async_copy(h_ref.at[sv], hb, sem_h)
        off = c * e_total + tile_e0 + k * EDGE_BLK
        pltpu.async_copy(e_ref.at[pl.ds(off, EDGE_BLK)], eb, sem_e)

    def _wait_data(sv, hb, eb):
        pltpu.make_async_copy(h_ref.at[sv], hb, sem_h).wait()
        pltpu.make_async_copy(e_ref.at[pl.ds(0, EDGE_BLK)], eb, sem_e).wait()

    def _compute(hb, eb, vb):
        def _row(i, _):
            for u in range(4):
                for k2 in range(HH // 16):
                    hv = hb[4 * i + u, pl.ds(k2 * 16, 16)]
                    ev = eb[4 * i + u, pl.ds(k2 * 16, 16)]
                    m = jnp.maximum(hv + ev, 0.0)
                    x = jnp.exp(m)
                    vb[4 * i + u, pl.ds(k2 * 16, 16)] = x * m
                    vb[4 * i + u, pl.ds(HH + k2 * 16, 16)] = x
            return 0
        lax.fori_loop(0, EDGE_BLK // 4, _row, 0)

    def _wait_scatter(vb):
        pltpu.make_async_copy(vb, accum.at[dstv0], sem_s).wait()

    pltpu.sync_copy(src_ref.at[pl.ds(tile_e0, EDGE_BLK)], srcv0)
    pltpu.sync_copy(dst_ref.at[pl.ds(tile_e0, EDGE_BLK)], dstv0)
    _adjust(srcv0)
    _issue_data(0, srcv0, hrows0, erows0)
    _issue_idx(1, srcv1, dstv1)

    srcs = (srcv0, srcv1)
    dsts = (dstv0, dstv1, dstv2, dstv3)
    hrs = (hrows0, hrows1)
    ers = (erows0, erows1)
    vls = (vals0, vals1)

    def _slot(k, p, d):
        @pl.when(k < nblk)
        def _():
            @pl.when(k + 1 < nblk)
            def _():
                _wait_idx(srcs[p ^ 1], dsts[(d + 1) % 4])
                _adjust(srcs[p ^ 1])
                _issue_data(k + 1, srcs[p ^ 1], hrs[p ^ 1], ers[p ^ 1])

            _wait_data(srcs[p], hrs[p], ers[p])

            @pl.when(k >= 2)
            def _():
                _wait_scatter(vls[p])

            _compute(hrs[p], ers[p], vls[p])
            pltpu.async_copy(vls[p], accum.at[dsts[d]], sem_s, add=True)

            @pl.when(k + 2 < nblk)
            def _():
                _issue_idx(k + 2, srcs[p], dsts[(d + 2) % 4])

    def _quad(i, _):
        for u in range(4):
            _slot(4 * i + u, u % 2, u)
        return 0
    lax.fori_loop(0, (nblk + 3) // 4, _quad, 0)

    _wait_scatter(vls[0])
    _wait_scatter(vls[1])
    plsc.subcore_barrier()
    pltpu.sync_copy(accum.at[pl.ds(s * rows_per_tile, rows_per_tile)],
                    out_ref.at[pl.ds(c * NPAD + s * rows_per_tile,
                                     rows_per_tile)])


def _edge_pass(h_split, e2, src, dst, n):
    h2 = h_split.reshape(2 * n, HH)
    mesh = plsc.VectorSubcoreMesh(core_axis_name="c", subcore_axis_name="s")
    out = pl.kernel(
        _edge_sc_body,
        mesh=mesh,
        compiler_params=pltpu.CompilerParams(use_tc_tiling_on_sc=False),
        out_type=jax.ShapeDtypeStruct((2 * NPAD, H), F32),
        scratch_types=[
            pltpu.VMEM_SHARED((NPAD, H), F32),
            pltpu.VMEM((EDGE_BLK,), jnp.int32),
            pltpu.VMEM((EDGE_BLK,), jnp.int32),
            pltpu.VMEM((EDGE_BLK,), jnp.int32),
            pltpu.VMEM((EDGE_BLK,), jnp.int32),
            pltpu.VMEM((EDGE_BLK,), jnp.int32),
            pltpu.VMEM((EDGE_BLK,), jnp.int32),
            pltpu.VMEM((EDGE_BLK, HH), F32),
            pltpu.VMEM((EDGE_BLK, HH), F32),
            pltpu.VMEM((EDGE_BLK, HH), F32),
            pltpu.VMEM((EDGE_BLK, HH), F32),
            pltpu.VMEM((EDGE_BLK, H), F32),
            pltpu.VMEM((EDGE_BLK, H), F32),
            pltpu.SemaphoreType.DMA,
            pltpu.SemaphoreType.DMA,
            pltpu.SemaphoreType.DMA,
            pltpu.SemaphoreType.DMA,
        ],
    )(h2, e2, src, dst)
    return out.reshape(2, NPAD, H)


def _pool(x, batch):
    s = jax.ops.segment_sum(x, batch, num_segments=NGRAPH)
    cnt = jax.ops.segment_sum(jnp.ones((x.shape[0],), F32), batch,
                              num_segments=NGRAPH)
    cntb = jnp.broadcast_to(cnt[:, None], (NGRAPH, H))
    part = jnp.concatenate([s, cntb], axis=1)
    return jnp.stack([part, jnp.zeros_like(part)])


def kernel(entry1_x, entry1_edge_index, entry1_edge_attr, entry1_batch,
           entry2_x, entry2_edge_index, entry2_edge_attr, entry2_batch,
           node_W, node_b, edge_W, edge_b, ln_g, ln_b, mlp_W, mlp_b,
           fc1_W, fc1_b, bn1_g, bn1_b, fc2_W, fc2_b, bn2_g, bn2_b,
           out_W, out_b):
    num_layers = mlp_W.shape[0]
    n = entry1_x.shape[0]
    e_num = entry1_edge_index.shape[1]
    src1, dst1 = entry1_edge_index[0], entry1_edge_index[1]
    src2, dst2 = entry2_edge_index[0], entry2_edge_index[1]
    e1 = _edge_proj(entry1_edge_attr, edge_W, edge_b).reshape(2 * e_num, HH)
    e2 = _edge_proj(entry2_edge_attr, edge_W, edge_b).reshape(2 * e_num, HH)
    x1, h1 = _proj(entry1_x, node_W, node_b, ln_g[0], ln_b[0])
    x2, h2 = _proj(entry2_x, node_W, node_b, ln_g[0], ln_b[0])
    for l in range(num_layers):
        a1 = _edge_pass(h1, e1, src1, dst1, n)
        a2 = _edge_pass(h2, e2, src2, dst2, n)
        nl = min(l + 1, num_layers - 1)
        w2 = mlp_W[l].reshape(2, HH, H)
        x1, h1 = _dense_step(x1, a1, w2, mlp_b[l], ln_g[nl], ln_b[nl])
        x2, h2 = _dense_step(x2, a2, w2, mlp_b[l], ln_g[nl], ln_b[nl])
    p1 = _pool(x1, entry1_batch)
    p2 = _pool(x2, entry2_batch)
    return _head(p1, p2, fc1_W, fc1_b, bn1_g, bn1_b, fc2_W, fc2_b,
                 bn2_g, bn2_b, out_W, out_b)
